# baseline (speedup 1.0000x reference)
import functools

import jax
import jax.numpy as jnp
from jax import lax
from jax.experimental import pallas as pl
from jax.experimental.pallas import tpu as pltpu
from jax.experimental.pallas import tpu_sc as plsc

_EPS = 1e-5


def _sc_gather_t(m, cat_t, f_per_row):
    R, V = m.shape
    F, B = cat_t.shape
    info = plsc.get_sparse_core_info()
    nw = info.num_cores * info.num_subcores
    rows_per_w = R // nw
    out_chunk = 4096
    n_chunks = B // out_chunk
    unroll = 8
    mesh = plsc.VectorSubcoreMesh(core_axis_name="core",
                                  subcore_axis_name="subcore")

    @functools.partial(
        pl.kernel,
        out_type=jax.ShapeDtypeStruct((R, B), jnp.float32),
        mesh=mesh,
        compiler_params=pltpu.CompilerParams(needs_layout_passes=False),
        scratch_types=[
            pltpu.VMEM((V,), jnp.float32),
            pltpu.VMEM((B,), jnp.int32),
            pltpu.VMEM((out_chunk,), jnp.float32),
            pltpu.VMEM((out_chunk,), jnp.float32),
            pltpu.SemaphoreType.DMA,
            pltpu.SemaphoreType.DMA,
            pltpu.SemaphoreType.DMA,
        ],
    )
    def k(m_hbm, cat_hbm, out_hbm, row_v, idx_v, ob0, ob1, sem_row,
          sem_o0, sem_o1):
        wid = (lax.axis_index("subcore") * info.num_cores
               + lax.axis_index("core"))
        row0 = wid * rows_per_w
        obufs = (ob0, ob1)
        osems = (sem_o0, sem_o1)

        pltpu.make_async_copy(m_hbm.at[row0], row_v, sem_row).start()

        @pl.loop(0, rows_per_w)
        def _(j):
            r = row0 + j

            @pl.when(jnp.logical_or(j == 0, lax.rem(r, f_per_row) == 0))
            def _():
                pltpu.sync_copy(cat_hbm.at[r // f_per_row], idx_v)

            pltpu.make_async_copy(m_hbm.at[r], row_v, sem_row).wait()

            for c in range(n_chunks):
                ob = obufs[c % 2]
                osem = osems[c % 2]
                if c >= 2:
                    pltpu.make_async_copy(
                        ob, out_hbm.at[0, pl.ds(0, out_chunk)], osem).wait()
                else:
                    @pl.when(j > 0)
                    def _():
                        pltpu.make_async_copy(
                            ob, out_hbm.at[0, pl.ds(0, out_chunk)],
                            osem).wait()

                @pl.loop(0, out_chunk // 16, step=unroll)
                def _(t):
                    for u in range(unroll):
                        iv = idx_v[pl.ds(c * out_chunk + (t + u) * 16, 16)]
                        ob[pl.ds((t + u) * 16, 16)] = plsc.load_gather(
                            row_v, [iv])

                if c == n_chunks - 1:
                    @pl.when(j + 1 < rows_per_w)
                    def _():
                        pltpu.make_async_copy(m_hbm.at[r + 1], row_v,
                                              sem_row).start()
                pltpu.make_async_copy(
                    ob, out_hbm.at[r, pl.ds(c * out_chunk, out_chunk)],
                    osem).start()

        for p in range(2):
            pltpu.make_async_copy(obufs[p],
                                  out_hbm.at[0, pl.ds(0, out_chunk)],
                                  osems[p]).wait()

    return k(m, cat_t)


def _fused_body(embt_ref, numt_ref, num_ref, w1e_ref, w1n_ref, b1_ref,
                g0_ref, be0_ref, w2_ref, b2_ref, g1_ref, be1_ref,
                wo_ref, g2_ref, be2_ref, bo_ref, o_ref,
                x1_s, x2_s, st1_s, st2_s, *, tile, n_rows):
    p = pl.program_id(0)
    i = pl.program_id(1)
    rows = pl.ds(i * tile, tile)
    def col_stats(st_s, x):
        @pl.when(i == 0)
        def _():
            st_s[...] = jnp.zeros_like(st_s)

        st_s[...] += jnp.concatenate(
            [jnp.sum(x, axis=0, keepdims=True),
             jnp.sum(x * x, axis=0, keepdims=True)], axis=0)

    def bn_fold(st_s, g_ref, be_ref):
        m = st_s[0:1, :] / n_rows
        v = st_s[1:2, :] / n_rows - m * m
        scale = g_ref[...] * lax.rsqrt(v + _EPS)
        shift = be_ref[...] - m * scale
        return scale, shift

    @pl.when(p == 0)
    def _():
        numt = numt_ref[...]
        m0 = jnp.sum(numt, axis=1, keepdims=True) / n_rows
        v0 = jnp.sum(numt * numt, axis=1, keepdims=True) / n_rows - m0 * m0
        scale0 = g0_ref[...] * lax.rsqrt(v0 + _EPS)
        shift0 = be0_ref[...] - m0 * scale0
        w1n = w1n_ref[...]
        bias = b1_ref[...] + jnp.sum(w1n * shift0, axis=0, keepdims=True)

        x1 = lax.dot_general(embt_ref[...].astype(jnp.bfloat16),
                             w1e_ref[...].astype(jnp.bfloat16),
                             (((0,), (0,)), ((), ())),
                             preferred_element_type=jnp.float32)
        x1 = x1 + jnp.dot(num_ref[...], w1n * scale0,
                          preferred_element_type=jnp.float32)
        x1 = jnp.maximum(x1 + bias, 0.0)
        col_stats(st1_s, x1)
        x1_s[rows, :] = x1

    @pl.when(p == 1)
    def _():
        scale1, shift1 = bn_fold(st1_s, g1_ref, be1_ref)
        xn = x1_s[rows, :] * scale1 + shift1
        x2 = jnp.dot(xn.astype(jnp.bfloat16),
                     w2_ref[...].astype(jnp.bfloat16),
                     preferred_element_type=jnp.float32)
        x2 = jnp.maximum(x2 + b2_ref[...], 0.0)
        col_stats(st2_s, x2)
        x2_s[rows, :] = x2.astype(jnp.bfloat16)

    @pl.when(p == 2)
    def _():
        scale2, shift2 = bn_fold(st2_s, g2_ref, be2_ref)
        xn = x2_s[rows, :].astype(jnp.float32) * scale2 + shift2
        o_ref[...] = jnp.dot(xn, wo_ref[...],
                             preferred_element_type=jnp.float32) + bo_ref[...]


def _mlp(embt, numerical_data, W1, b1, W2, b2, Wo, bo,
         g0, be0, g1, be1, g2, be2, tile):
    B, NUM = numerical_data.shape
    E = embt.shape[0]
    H1 = W1.shape[0]
    H2 = W2.shape[0]
    nb = B // tile
    const = lambda p, i: (0, 0)

    return pl.pallas_call(
        functools.partial(_fused_body, tile=tile, n_rows=float(B)),
        grid=(3, nb),
        in_specs=[
            pl.BlockSpec((E, tile), lambda p, i: (0, jnp.where(p == 0, i, 0))),
            pl.BlockSpec((NUM, B), const),
            pl.BlockSpec((tile, NUM),
                         lambda p, i: (jnp.where(p == 0, i, 0), 0)),
            pl.BlockSpec((E, H1), const),
            pl.BlockSpec((NUM, H1), const),
            pl.BlockSpec((1, H1), const),
            pl.BlockSpec((NUM, 1), const),
            pl.BlockSpec((NUM, 1), const),
            pl.BlockSpec((H1, H2), const),
            pl.BlockSpec((1, H2), const),
            pl.BlockSpec((1, H1), const),
            pl.BlockSpec((1, H1), const),
            pl.BlockSpec((H2, 1), const),
            pl.BlockSpec((1, H2), const),
            pl.BlockSpec((1, H2), const),
            pl.BlockSpec((1, 1), const),
        ],
        out_specs=pl.BlockSpec((tile, 1),
                               lambda p, i: (jnp.where(p == 2, i, 0), 0)),
        out_shape=jax.ShapeDtypeStruct((B, 1), jnp.float32),
        scratch_shapes=[
            pltpu.VMEM((B, H1), jnp.float32),
            pltpu.VMEM((B, H2), jnp.bfloat16),
            pltpu.VMEM((2, H1), jnp.float32),
            pltpu.VMEM((2, H2), jnp.float32),
        ],
    )(embt, numerical_data.T, numerical_data, W1[:, :E].T, W1[:, E:].T,
      b1[None, :], g0[:, None], be0[:, None], W2.T, b2[None, :],
      g1[None, :], be1[None, :], Wo.T, g2[None, :], be2[None, :],
      bo[None, :])


def kernel(numerical_data, cat_data, tables, W1, b1, W2, b2, Wo, bo,
           g0, be0, g1, be1, g2, be2):
    B, NUM = numerical_data.shape
    F, V, D = tables.shape
    m = tables.transpose(0, 2, 1).reshape(F * D, V)
    cat_t = cat_data.T.astype(jnp.int32)
    embt = _sc_gather_t(m, cat_t, f_per_row=D)
    return _mlp(embt, numerical_data, W1, b1, W2, b2, Wo, bo,
                g0, be0, g1, be1, g2, be2, tile=1024)


# speedup vs baseline: 1.0473x; 1.0473x over previous
import functools

import jax
import jax.numpy as jnp
from jax import lax
from jax.experimental import pallas as pl
from jax.experimental.pallas import tpu as pltpu
from jax.experimental.pallas import tpu_sc as plsc

_EPS = 1e-5


def _sc_gather_t(m, cat_t, f_per_row):
    R, V = m.shape
    F, B = cat_t.shape
    info = plsc.get_sparse_core_info()
    nw = info.num_cores * info.num_subcores
    rows_per_w = R // nw
    out_chunk = 4096
    n_chunks = B // out_chunk
    unroll = 8
    mesh = plsc.VectorSubcoreMesh(core_axis_name="core",
                                  subcore_axis_name="subcore")

    @functools.partial(
        pl.kernel,
        out_type=jax.ShapeDtypeStruct((R, B), jnp.float32),
        mesh=mesh,
        compiler_params=pltpu.CompilerParams(needs_layout_passes=False),
        scratch_types=[
            pltpu.VMEM((V,), jnp.float32),
            pltpu.VMEM((B,), jnp.int32),
            pltpu.VMEM((out_chunk,), jnp.float32),
            pltpu.VMEM((out_chunk,), jnp.float32),
            pltpu.SemaphoreType.DMA,
            pltpu.SemaphoreType.DMA,
            pltpu.SemaphoreType.DMA,
        ],
    )
    def k(m_hbm, cat_hbm, out_hbm, row_v, idx_v, ob0, ob1, sem_row,
          sem_o0, sem_o1):
        wid = (lax.axis_index("subcore") * info.num_cores
               + lax.axis_index("core"))
        row0 = wid * rows_per_w
        obufs = (ob0, ob1)
        osems = (sem_o0, sem_o1)

        pltpu.make_async_copy(m_hbm.at[row0], row_v, sem_row).start()

        @pl.loop(0, rows_per_w)
        def _(j):
            r = row0 + j

            @pl.when(jnp.logical_or(j == 0, lax.rem(r, f_per_row) == 0))
            def _():
                pltpu.sync_copy(cat_hbm.at[r // f_per_row], idx_v)

            pltpu.make_async_copy(m_hbm.at[r], row_v, sem_row).wait()

            for c in range(n_chunks):
                ob = obufs[c % 2]
                osem = osems[c % 2]
                if c >= 2:
                    pltpu.make_async_copy(
                        ob, out_hbm.at[0, pl.ds(0, out_chunk)], osem).wait()
                else:
                    @pl.when(j > 0)
                    def _():
                        pltpu.make_async_copy(
                            ob, out_hbm.at[0, pl.ds(0, out_chunk)],
                            osem).wait()

                @pl.loop(0, out_chunk // 16, step=unroll)
                def _(t):
                    for u in range(unroll):
                        iv = idx_v[pl.ds(c * out_chunk + (t + u) * 16, 16)]
                        ob[pl.ds((t + u) * 16, 16)] = plsc.load_gather(
                            row_v, [iv])

                if c == n_chunks - 1:
                    @pl.when(j + 1 < rows_per_w)
                    def _():
                        pltpu.make_async_copy(m_hbm.at[r + 1], row_v,
                                              sem_row).start()
                pltpu.make_async_copy(
                    ob, out_hbm.at[r, pl.ds(c * out_chunk, out_chunk)],
                    osem).start()

        for p in range(2):
            pltpu.make_async_copy(obufs[p],
                                  out_hbm.at[0, pl.ds(0, out_chunk)],
                                  osems[p]).wait()

    return k(m, cat_t)


def _fused_body(embt_ref, numt_ref, num_ref, w1e_ref, w1n_ref, b1_ref,
                g0_ref, be0_ref, w2_ref, b2_ref, g1_ref, be1_ref,
                wo_ref, g2_ref, be2_ref, bo_ref, o_ref,
                x1_s, x2_s, st1_s, st2_s, *, tile, n_rows):
    p = pl.program_id(0)
    i = pl.program_id(1)
    rows = pl.ds(i * tile, tile)
    def col_stats(st_s, x):
        @pl.when(i == 0)
        def _():
            st_s[...] = jnp.zeros_like(st_s)

        st_s[...] += jnp.concatenate(
            [jnp.sum(x, axis=0, keepdims=True),
             jnp.sum(x * x, axis=0, keepdims=True)], axis=0)

    def bn_fold(st_s, g_ref, be_ref):
        m = st_s[0:1, :] / n_rows
        v = st_s[1:2, :] / n_rows - m * m
        scale = g_ref[...] * lax.rsqrt(v + _EPS)
        shift = be_ref[...] - m * scale
        return scale, shift

    @pl.when(p == 0)
    def _():
        numt = numt_ref[...]
        m0 = jnp.sum(numt, axis=1, keepdims=True) / n_rows
        v0 = jnp.sum(numt * numt, axis=1, keepdims=True) / n_rows - m0 * m0
        scale0 = g0_ref[...] * lax.rsqrt(v0 + _EPS)
        shift0 = be0_ref[...] - m0 * scale0
        w1n = w1n_ref[...]
        bias = b1_ref[...] + jnp.sum(w1n * shift0, axis=0, keepdims=True)

        x1 = lax.dot_general(embt_ref[...].astype(jnp.bfloat16),
                             w1e_ref[...].astype(jnp.bfloat16),
                             (((0,), (0,)), ((), ())),
                             preferred_element_type=jnp.float32)
        x1 = x1 + jnp.dot(num_ref[...], w1n * scale0,
                          preferred_element_type=jnp.float32)
        x1 = jnp.maximum(x1 + bias, 0.0)
        col_stats(st1_s, x1)
        x1_s[rows, :] = x1.astype(jnp.bfloat16)

    @pl.when(p == 1)
    def _():
        scale1, shift1 = bn_fold(st1_s, g1_ref, be1_ref)
        xn = x1_s[rows, :].astype(jnp.float32) * scale1 + shift1
        x2 = jnp.dot(xn.astype(jnp.bfloat16),
                     w2_ref[...].astype(jnp.bfloat16),
                     preferred_element_type=jnp.float32)
        x2 = jnp.maximum(x2 + b2_ref[...], 0.0)
        col_stats(st2_s, x2)
        x2_s[rows, :] = x2

    @pl.when(p == 2)
    def _():
        scale2, shift2 = bn_fold(st2_s, g2_ref, be2_ref)
        xn = x2_s[rows, :] * scale2 + shift2
        o_ref[...] = jnp.dot(xn, wo_ref[...],
                             preferred_element_type=jnp.float32) + bo_ref[...]


def _mlp(embt, numerical_data, W1, b1, W2, b2, Wo, bo,
         g0, be0, g1, be1, g2, be2, tile):
    B, NUM = numerical_data.shape
    E = embt.shape[0]
    H1 = W1.shape[0]
    H2 = W2.shape[0]
    nb = B // tile
    const = lambda p, i: (0, 0)

    return pl.pallas_call(
        functools.partial(_fused_body, tile=tile, n_rows=float(B)),
        grid=(3, nb),
        in_specs=[
            pl.BlockSpec((E, tile), lambda p, i: (0, jnp.where(p == 0, i, 0))),
            pl.BlockSpec((NUM, B), const),
            pl.BlockSpec((tile, NUM),
                         lambda p, i: (jnp.where(p == 0, i, 0), 0)),
            pl.BlockSpec((E, H1), const),
            pl.BlockSpec((NUM, H1), const),
            pl.BlockSpec((1, H1), const),
            pl.BlockSpec((NUM, 1), const),
            pl.BlockSpec((NUM, 1), const),
            pl.BlockSpec((H1, H2), const),
            pl.BlockSpec((1, H2), const),
            pl.BlockSpec((1, H1), const),
            pl.BlockSpec((1, H1), const),
            pl.BlockSpec((H2, 1), const),
            pl.BlockSpec((1, H2), const),
            pl.BlockSpec((1, H2), const),
            pl.BlockSpec((1, 1), const),
        ],
        out_specs=pl.BlockSpec((tile, 1),
                               lambda p, i: (jnp.where(p == 2, i, 0), 0)),
        out_shape=jax.ShapeDtypeStruct((B, 1), jnp.float32),
        scratch_shapes=[
            pltpu.VMEM((B, H1), jnp.bfloat16),
            pltpu.VMEM((B, H2), jnp.float32),
            pltpu.VMEM((2, H1), jnp.float32),
            pltpu.VMEM((2, H2), jnp.float32),
        ],
    )(embt, numerical_data.T, numerical_data, W1[:, :E].T, W1[:, E:].T,
      b1[None, :], g0[:, None], be0[:, None], W2.T, b2[None, :],
      g1[None, :], be1[None, :], Wo.T, g2[None, :], be2[None, :],
      bo[None, :])


def kernel(numerical_data, cat_data, tables, W1, b1, W2, b2, Wo, bo,
           g0, be0, g1, be1, g2, be2):
    B, NUM = numerical_data.shape
    F, V, D = tables.shape
    m = tables.transpose(0, 2, 1).reshape(F * D, V)
    cat_t = cat_data.T.astype(jnp.int32)
    embt = _sc_gather_t(m, cat_t, f_per_row=D)
    return _mlp(embt, numerical_data, W1, b1, W2, b2, Wo, bo,
                g0, be0, g1, be1, g2, be2, tile=2048)
